# baseline (speedup 1.0000x reference)
import functools

import jax
import jax.numpy as jnp
from jax import lax
from jax.experimental import pallas as pl
from jax.experimental.pallas import tpu as pltpu
from jax.experimental.pallas import tpu_sc as plsc

VOCAB, EMBED, BATCH, SEQ = 1000, 64, 4096, 200
NC, NS = 2, 16
NW = NC * NS
BB = BATCH // NW
LANES = 16
KD = EMBED // LANES
DT = EMBED // 8

_mesh = plsc.VectorSubcoreMesh(
    core_axis_name="c", subcore_axis_name="s", num_cores=NC, num_subcores=NS
)


@functools.partial(
    pl.kernel,
    out_type=jax.ShapeDtypeStruct((SEQ, DT, NW, 8, BB), jnp.float32),
    mesh=_mesh,
    scratch_types=[
        pltpu.VMEM((SEQ, BB), jnp.int32),
        pltpu.VMEM((SEQ, EMBED), jnp.float32),
        pltpu.VMEM((BB, EMBED), jnp.float32),
        pltpu.VMEM((DT, 8, BB), jnp.float32),
        pltpu.SemaphoreType.DMA,
        pltpu.SemaphoreType.DMA,
    ],
    compiler_params=pltpu.CompilerParams(use_tc_tiling_on_sc=False,
                                         needs_layout_passes=False),
)
def _embed_kernel(xt_hbm, char_hbm, pos_hbm, out_hbm,
                  xbuf, posb, gbuf, tbuf, gsem, osem):
    wid = lax.axis_index("s") * NC + lax.axis_index("c")
    b0 = wid * BB
    pltpu.sync_copy(xt_hbm.at[:, pl.ds(b0, BB)], xbuf)
    pltpu.sync_copy(pos_hbm, posb)

    iota = jnp.arange(LANES, dtype=jnp.int32)

    def step(s, carry):
        pltpu.async_copy(char_hbm.at[xbuf.at[s]], gbuf, gsem).wait()

        pv = [posb[s, pl.ds(k * LANES, LANES)] for k in range(KD)]

        def add_body(b, c2):
            for k in range(KD):
                plsc.addupdate(gbuf.at[b, pl.ds(k * LANES, LANES)], pv[k])
            return c2

        lax.fori_loop(0, BB, add_body, 0)

        for d in range(EMBED):
            dd = jnp.full((LANES,), d, jnp.int32)
            trow = tbuf.at[d // 8, d % 8]
            for j in range(BB // LANES):
                v = plsc.load_gather(gbuf, [iota + j * LANES, dd])
                trow[pl.ds(j * LANES, LANES)] = v

        pltpu.sync_copy(tbuf, out_hbm.at[s, :, wid])
        return carry

    lax.fori_loop(0, SEQ, step, 0)


def kernel(input_x, char_table, pos_table):
    x_t = input_x.T
    out4 = _embed_kernel(x_t, char_table, pos_table)
    return out4.transpose(2, 4, 0, 1, 3).reshape(BATCH, SEQ, EMBED)


# speedup vs baseline: 1.2180x; 1.2180x over previous
import functools

import jax
import jax.numpy as jnp
from jax import lax
from jax.experimental import pallas as pl
from jax.experimental.pallas import tpu as pltpu
from jax.experimental.pallas import tpu_sc as plsc

VOCAB, EMBED, BATCH, SEQ = 1000, 64, 4096, 200
NC, NS = 2, 16
NW = NC * NS
BB = BATCH // NW
LANES = 16
KD = EMBED // LANES
DT = EMBED // 8
GB = EMBED + 1

_mesh = plsc.VectorSubcoreMesh(
    core_axis_name="c", subcore_axis_name="s", num_cores=NC, num_subcores=NS
)


@functools.partial(
    pl.kernel,
    out_type=jax.ShapeDtypeStruct((SEQ, DT, NW, 8, BB), jnp.float32),
    mesh=_mesh,
    scratch_types=[
        pltpu.VMEM((SEQ, BB), jnp.int32),
        pltpu.VMEM((SEQ, EMBED), jnp.float32),
        pltpu.VMEM((BB, EMBED), jnp.float32),
        pltpu.VMEM((BB, GB), jnp.float32),
        pltpu.VMEM((DT, 8, BB), jnp.float32),
        pltpu.SemaphoreType.DMA,
        pltpu.SemaphoreType.DMA,
    ],
    compiler_params=pltpu.CompilerParams(use_tc_tiling_on_sc=False,
                                         needs_layout_passes=False),
)
def _embed_kernel(xt_hbm, char_hbm, pos_hbm, out_hbm,
                  xbuf, posb, gbuf, rbuf, tbuf, gsem, osem):
    wid = lax.axis_index("s") * NC + lax.axis_index("c")
    b0 = wid * BB
    pltpu.sync_copy(xt_hbm.at[:, pl.ds(b0, BB)], xbuf)
    pltpu.sync_copy(pos_hbm, posb)

    iota = jnp.arange(LANES, dtype=jnp.int32)

    def step(s, carry):
        pltpu.async_copy(char_hbm.at[xbuf.at[s]], gbuf, gsem).wait()

        pv = [posb[s, pl.ds(k * LANES, LANES)] for k in range(KD)]

        def add_body(b, c2):
            row = rbuf.at[b]
            for k in range(KD):
                row[pl.ds(k * LANES, LANES)] = (
                    gbuf[b, pl.ds(k * LANES, LANES)] + pv[k])
            return c2

        lax.fori_loop(0, BB, add_body, 0)

        for d in range(EMBED):
            dd = jnp.full((LANES,), d, jnp.int32)
            trow = tbuf.at[d // 8, d % 8]
            for j in range(BB // LANES):
                v = plsc.load_gather(rbuf, [iota + j * LANES, dd])
                trow[pl.ds(j * LANES, LANES)] = v

        pltpu.sync_copy(tbuf, out_hbm.at[s, :, wid])
        return carry

    lax.fori_loop(0, SEQ, step, 0)


def kernel(input_x, char_table, pos_table):
    x_t = input_x.T
    out4 = _embed_kernel(x_t, char_table, pos_table)
    return out4.transpose(2, 4, 0, 1, 3).reshape(BATCH, SEQ, EMBED)


# speedup vs baseline: 1.9357x; 1.5892x over previous
import functools

import jax
import jax.numpy as jnp
from jax import lax
from jax.experimental import pallas as pl
from jax.experimental.pallas import tpu as pltpu
from jax.experimental.pallas import tpu_sc as plsc

VOCAB, EMBED, BATCH, SEQ = 1000, 64, 4096, 200
NC, NS = 2, 16
NW = NC * NS
BB = BATCH // NW
LANES = 16
KD = EMBED // LANES
DT = EMBED // 8
GB = EMBED + 1

_mesh = plsc.VectorSubcoreMesh(
    core_axis_name="c", subcore_axis_name="s", num_cores=NC, num_subcores=NS
)


@functools.partial(
    pl.kernel,
    out_type=jax.ShapeDtypeStruct((SEQ, DT, NW, 8, BB), jnp.float32),
    mesh=_mesh,
    scratch_types=[
        pltpu.VMEM((SEQ, BB), jnp.int32),
        pltpu.VMEM((SEQ, EMBED), jnp.float32),
        pltpu.VMEM((BB, EMBED), jnp.float32),
        pltpu.VMEM((BB, EMBED), jnp.float32),
        pltpu.VMEM((BB, GB), jnp.float32),
        pltpu.VMEM((DT, 8, BB), jnp.float32),
        pltpu.VMEM((DT, 8, BB), jnp.float32),
        pltpu.SemaphoreType.DMA,
        pltpu.SemaphoreType.DMA,
        pltpu.SemaphoreType.DMA,
        pltpu.SemaphoreType.DMA,
    ],
    compiler_params=pltpu.CompilerParams(use_tc_tiling_on_sc=False,
                                         needs_layout_passes=False),
)
def _embed_kernel(xt_hbm, char_hbm, pos_hbm, out_hbm,
                  xbuf, posb, gbuf0, gbuf1, rbuf, tbuf0, tbuf1,
                  gsem0, gsem1, osem0, osem1):
    wid = lax.axis_index("s") * NC + lax.axis_index("c")
    b0 = wid * BB
    pltpu.sync_copy(xt_hbm.at[:, pl.ds(b0, BB)], xbuf)
    pltpu.sync_copy(pos_hbm, posb)

    gbuf = (gbuf0, gbuf1)
    tbuf = (tbuf0, tbuf1)
    gsem = (gsem0, gsem1)
    osem = (osem0, osem1)
    iota = jnp.arange(LANES, dtype=jnp.int32)

    pltpu.async_copy(char_hbm.at[xbuf.at[0]], gbuf0, gsem0)

    def step(s, t):
        @pl.when(s + 1 < SEQ)
        def _():
            pltpu.async_copy(char_hbm.at[xbuf.at[s + 1]], gbuf[1 - t],
                             gsem[1 - t])

        pltpu.make_async_copy(char_hbm.at[xbuf.at[s]], gbuf[t],
                              gsem[t]).wait()

        pv = [posb[s, pl.ds(k * LANES, LANES)] for k in range(KD)]

        @functools.partial(plsc.parallel_loop, 0, BB, unroll=4)
        def _(b):
            row = rbuf.at[b]
            for k in range(KD):
                row[pl.ds(k * LANES, LANES)] = (
                    gbuf[t][b, pl.ds(k * LANES, LANES)] + pv[k])

        for d in range(EMBED):
            dd = jnp.full((LANES,), d, jnp.int32)
            trow = tbuf[t].at[d // 8, d % 8]
            for j in range(BB // LANES):
                v = plsc.load_gather(rbuf, [iota + j * LANES, dd])
                trow[pl.ds(j * LANES, LANES)] = v

        pltpu.sync_copy(tbuf[t], out_hbm.at[s, :, wid])

    def body(s2, carry):
        step(s2, 0)
        step(s2 + 1, 1)
        return carry

    lax.fori_loop(0, SEQ // 2, lambda i, c: body(i * 2, c), 0)


def kernel(input_x, char_table, pos_table):
    x_t = input_x.T
    out4 = _embed_kernel(x_t, char_table, pos_table)
    return out4.transpose(2, 4, 0, 1, 3).reshape(BATCH, SEQ, EMBED)
